# baseline (speedup 1.0000x reference)
import jax
import jax.numpy as jnp
from jax import lax
from jax.experimental import pallas as pl
from jax.experimental.pallas import tpu as pltpu
from jax.experimental.pallas import tpu_sc as plsc

N = 10000
E = 320000
H = 128
NT = 16
L = 3

NC = 2
NS = 16
NW = NC * NS
C = 128
NCH = 80
EPW = NCH * C
EP = NW * EPW
NP2 = 10240
NPAD = N + 16

_MESH = plsc.VectorSubcoreMesh(
    core_axis_name="c", subcore_axis_name="s", num_cores=NC, num_subcores=NS)


GRP = 16
NGRP = NCH // GRP
LCH = (48, 112)
ECH = (64, 96)


def _layer_sc_body(src_hbm, dst_hbm, et_hbm, h_hbm, asn_hbm, adn_hbm, att_hbm,
                   s_out, agg_out,
                   src_g, dst_g, et_g, att_v, asn0, adn0, asn1, adn1,
                   ex0, ex1, row0, row1,
                   zbuf, zvec, shared_s, shared_agg, sem1, sem2):
    c = lax.axis_index("c")
    s = lax.axis_index("s")
    nch_c = jnp.where(c == 0, LCH[0], LCH[1])
    base = jnp.where(c == 0, s * LCH[0], NS * LCH[0] + s * LCH[1])

    for i in range(8):
        for k in range(8):
            zbuf[i, pl.ds(k * 16, 16)] = jnp.zeros((16,), jnp.float32)
    for i in range(40):
        zvec[pl.ds(i * 16, 16)] = jnp.zeros((16,), jnp.float32)

    def zagg(i, _):
        pltpu.sync_copy(zbuf, shared_agg.at[pl.ds(s * 640 + i * 8, 8), :])
        return _
    lax.fori_loop(0, 80, zagg, 0)
    pltpu.sync_copy(zvec, shared_s.at[pl.ds(s * 640, 640)])

    pltpu.sync_copy(att_hbm, att_v)

    plsc.subcore_barrier()

    def fire(jj, row, asn_c, adn_c, sem):
        pltpu.async_copy(h_hbm.at[src_g.at[jj]], row, sem)
        pltpu.async_copy(asn_hbm.at[src_g.at[jj]], asn_c, sem)
        pltpu.async_copy(adn_hbm.at[dst_g.at[jj]], adn_c, sem)

    def drain(row, asn_c, adn_c, sem):
        pltpu.make_async_copy(h_hbm.at[pl.ds(0, C)], row, sem).wait()
        pltpu.make_async_copy(asn_hbm.at[pl.ds(0, C)], asn_c, sem).wait()
        pltpu.make_async_copy(adn_hbm.at[pl.ds(0, C)], adn_c, sem).wait()

    def compute(jj, row, asn_c, adn_c, exv_b):
        for gg in range(C // 16):
            ti = et_g[jj, pl.ds(gg * 16, 16)]
            z = (asn_c[pl.ds(gg * 16, 16)] + adn_c[pl.ds(gg * 16, 16)]
                 + plsc.load_gather(att_v, [ti]))
            e = jnp.where(z >= 0.0, z, z * jnp.float32(0.2))
            exv_b[pl.ds(gg * 16, 16)] = jnp.exp(e)

        def gbody(gg, _):
            exvec = exv_b[pl.ds(gg * 16, 16)]
            for rr in range(16):
                v = jnp.full((16,), exvec[rr], jnp.float32)
                r = gg * 16 + rr
                for k in range(H // 16):
                    row[r, pl.ds(k * 16, 16)] = row[r, pl.ds(k * 16, 16)] * v
            return _
        lax.fori_loop(0, C // 16, gbody, 0)

        idxd = dst_g.at[jj]
        pltpu.sync_copy(row, shared_agg.at[idxd], add=True)
        pltpu.sync_copy(exv_b, shared_s.at[idxd], add=True)

    def group(g, _):
        pltpu.sync_copy(src_hbm.at[pl.ds(base + g * GRP, GRP), :], src_g)
        pltpu.sync_copy(dst_hbm.at[pl.ds(base + g * GRP, GRP), :], dst_g)
        pltpu.sync_copy(et_hbm.at[pl.ds(base + g * GRP, GRP), :], et_g)

        fire(0, row0, asn0, adn0, sem1)

        def pair(p, carry):
            fire(2 * p + 1, row1, asn1, adn1, sem2)
            drain(row0, asn0, adn0, sem1)
            compute(2 * p, row0, asn0, adn0, ex0)

            @pl.when(p < GRP // 2 - 1)
            def _fire_next():
                fire(2 * p + 2, row0, asn0, adn0, sem1)
            drain(row1, asn1, adn1, sem2)
            compute(2 * p + 1, row1, asn1, adn1, ex1)
            return carry
        lax.fori_loop(0, GRP // 2, pair, 0)
        return _
    lax.fori_loop(0, nch_c // GRP, group, 0)

    plsc.subcore_barrier()

    @pl.when(s < 15)
    def _():
        pltpu.sync_copy(shared_agg.at[pl.ds(s * 624, 624), :],
                        agg_out.at[c, pl.ds(s * 624, 624), :])

    @pl.when(s == 15)
    def _():
        pltpu.sync_copy(shared_agg.at[pl.ds(15 * 624, 640), :],
                        agg_out.at[c, pl.ds(15 * 624, 640), :])

    @pl.when(s == 0)
    def _():
        pltpu.sync_copy(shared_s, s_out.at[c])


_layer_sc = pl.kernel(
    _layer_sc_body,
    out_type=[
        jax.ShapeDtypeStruct((NC, NP2), jnp.float32),
        jax.ShapeDtypeStruct((NC, N, H), jnp.float32),
    ],
    mesh=_MESH,
    compiler_params=pltpu.CompilerParams(needs_layout_passes=False),
    scratch_types=[
        pltpu.VMEM((GRP, C), jnp.int32),
        pltpu.VMEM((GRP, C), jnp.int32),
        pltpu.VMEM((GRP, C), jnp.int32),
        pltpu.VMEM((128,), jnp.float32),
        pltpu.VMEM((C,), jnp.float32),
        pltpu.VMEM((C,), jnp.float32),
        pltpu.VMEM((C,), jnp.float32),
        pltpu.VMEM((C,), jnp.float32),
        pltpu.VMEM((C,), jnp.float32),
        pltpu.VMEM((C,), jnp.float32),
        pltpu.VMEM((C, H), jnp.float32),
        pltpu.VMEM((C, H), jnp.float32),
        pltpu.VMEM((8, H), jnp.float32),
        pltpu.VMEM((640,), jnp.float32),
        pltpu.VMEM_SHARED((NP2,), jnp.float32),
        pltpu.VMEM_SHARED((NP2, H), jnp.float32),
        pltpu.SemaphoreType.DMA,
        pltpu.SemaphoreType.DMA,
    ],
)


def _edge_sc_body(src_hbm, dst_hbm, et_hbm, ps_hbm, pd_hbm, ct_hbm, w2_hbm,
                  b2_hbm, out_hbm,
                  srcv, dstv, etv, ctv, w2v, b2v, ps0, pd0, ps1, pd1,
                  outv, sem1, sem2):
    c = lax.axis_index("c")
    s = lax.axis_index("s")
    nch_c = jnp.where(c == 0, ECH[0], ECH[1])
    base = jnp.where(c == 0, s * ECH[0], NS * ECH[0] + s * ECH[1])

    @pl.when(c == 0)
    def _():
        pltpu.sync_copy(src_hbm.at[pl.ds(base, ECH[0]), :],
                        srcv.at[pl.ds(0, ECH[0])])
        pltpu.sync_copy(dst_hbm.at[pl.ds(base, ECH[0]), :],
                        dstv.at[pl.ds(0, ECH[0])])
        pltpu.sync_copy(et_hbm.at[pl.ds(base, ECH[0]), :],
                        etv.at[pl.ds(0, ECH[0])])

    @pl.when(c == 1)
    def _():
        pltpu.sync_copy(src_hbm.at[pl.ds(base, ECH[1]), :], srcv)
        pltpu.sync_copy(dst_hbm.at[pl.ds(base, ECH[1]), :], dstv)
        pltpu.sync_copy(et_hbm.at[pl.ds(base, ECH[1]), :], etv)
    pltpu.sync_copy(ct_hbm, ctv)
    pltpu.sync_copy(w2_hbm, w2v)
    pltpu.sync_copy(b2_hbm, b2v)

    w2s = [w2v[pl.ds(k * 16, 16)] for k in range(H // 16)]
    b2vec = b2v[pl.ds(0, 16)]
    lanes = lax.broadcasted_iota(jnp.int32, (16,), 0)

    def fire(j, psb, pdb, sem):
        pltpu.async_copy(ps_hbm.at[srcv.at[j]], psb, sem)
        pltpu.async_copy(pd_hbm.at[dstv.at[j]], pdb, sem)

    def drain(psb, pdb, sem):
        pltpu.make_async_copy(ps_hbm.at[pl.ds(0, C), :], psb, sem).wait()
        pltpu.make_async_copy(pd_hbm.at[pl.ds(0, C), :], pdb, sem).wait()

    def compute(j, psb, pdb):
        def gbody(g, _):
            etvec = etv[j, pl.ds(g * 16, 16)]
            lvec = jnp.zeros((16,), jnp.float32)
            for rr in range(16):
                et_r = etvec[rr]
                r = g * 16 + rr
                acc = jnp.zeros((16,), jnp.float32)
                for k in range(H // 16):
                    t = (psb[r, pl.ds(k * 16, 16)]
                         + pdb[r, pl.ds(k * 16, 16)]
                         + ctv[et_r, pl.ds(k * 16, 16)])
                    t = jnp.maximum(t, 0.0)
                    acc = acc + t * w2s[k]
                lvec = jnp.where(lanes == rr, jnp.sum(acc), lvec)
            v = lvec + b2vec
            outv[pl.ds(j * C + g * 16, 16)] = 1.0 / (1.0 + jnp.exp(-v))
            return _
        lax.fori_loop(0, C // 16, gbody, 0)

    fire(0, ps0, pd0, sem1)

    def pair(p, carry):
        fire(2 * p + 1, ps1, pd1, sem2)
        drain(ps0, pd0, sem1)
        compute(2 * p, ps0, pd0)

        @pl.when(p < nch_c // 2 - 1)
        def _fire_next():
            fire(2 * p + 2, ps0, pd0, sem1)
        drain(ps1, pd1, sem2)
        compute(2 * p + 1, ps1, pd1)
        return carry
    lax.fori_loop(0, nch_c // 2, pair, 0)

    @pl.when(c == 0)
    def _():
        pltpu.sync_copy(outv.at[pl.ds(0, ECH[0] * C)],
                        out_hbm.at[pl.ds(base * C, ECH[0] * C)])

    @pl.when(c == 1)
    def _():
        pltpu.sync_copy(outv.at[pl.ds(0, ECH[1] * C)],
                        out_hbm.at[pl.ds(base * C, ECH[1] * C)])


_edge_sc = pl.kernel(
    _edge_sc_body,
    out_type=[jax.ShapeDtypeStruct((EP,), jnp.float32)],
    mesh=_MESH,
    compiler_params=pltpu.CompilerParams(needs_layout_passes=False),
    scratch_types=[
        pltpu.VMEM((max(ECH), C), jnp.int32),
        pltpu.VMEM((max(ECH), C), jnp.int32),
        pltpu.VMEM((max(ECH), C), jnp.int32),
        pltpu.VMEM((NT, H), jnp.float32),
        pltpu.VMEM((H,), jnp.float32),
        pltpu.VMEM((128,), jnp.float32),
        pltpu.VMEM((C, H), jnp.float32),
        pltpu.VMEM((C, H), jnp.float32),
        pltpu.VMEM((C, H), jnp.float32),
        pltpu.VMEM((C, H), jnp.float32),
        pltpu.VMEM((max(ECH) * C,), jnp.float32),
        pltpu.SemaphoreType.DMA,
        pltpu.SemaphoreType.DMA,
    ],
)


_BR = 1000
_NB = N // _BR


def _tc0_body(x_ref, wt_ref, b_ref, asv_ref, adv_ref, h_ref, asn_ref, adn_ref):
    h = jnp.dot(x_ref[...], wt_ref[...],
                preferred_element_type=jnp.float32) + b_ref[...]
    h_ref[...] = h
    asn_ref[...] = jnp.dot(h, asv_ref[...], preferred_element_type=jnp.float32)
    adn_ref[...] = jnp.dot(h, adv_ref[...], preferred_element_type=jnp.float32)


def _tc0(x, wt, b, asv, adv):
    return pl.pallas_call(
        _tc0_body,
        grid=(_NB,),
        in_specs=[
            pl.BlockSpec((_BR, H), lambda i: (i, 0)),
            pl.BlockSpec((H, H), lambda i: (0, 0)),
            pl.BlockSpec((1, H), lambda i: (0, 0)),
            pl.BlockSpec((H, 1), lambda i: (0, 0)),
            pl.BlockSpec((H, 1), lambda i: (0, 0)),
        ],
        out_specs=[
            pl.BlockSpec((_BR, H), lambda i: (i, 0)),
            pl.BlockSpec((_BR, 1), lambda i: (i, 0)),
            pl.BlockSpec((_BR, 1), lambda i: (i, 0)),
        ],
        out_shape=[
            jax.ShapeDtypeStruct((N, H), jnp.float32),
            jax.ShapeDtypeStruct((N, 1), jnp.float32),
            jax.ShapeDtypeStruct((N, 1), jnp.float32),
        ],
    )(x, wt, b, asv, adv)


def _tclayer_body(a0_ref, a1_ref, s0_ref, s1_ref, wt_ref, b_ref, asv_ref,
                  adv_ref, h_ref, asn_ref, adn_ref):
    inv = 1.0 / (s0_ref[...] + s1_ref[...] + jnp.float32(1e-16))
    x = (a0_ref[...] + a1_ref[...]) * inv
    h = jnp.dot(x, wt_ref[...], preferred_element_type=jnp.float32) + b_ref[...]
    h = jnp.maximum(h, 0.0)
    h_ref[...] = h
    asn_ref[...] = jnp.dot(h, asv_ref[...], preferred_element_type=jnp.float32)
    adn_ref[...] = jnp.dot(h, adv_ref[...], preferred_element_type=jnp.float32)


def _tclayer(a0, a1, s0, s1, wt, b, asv, adv):
    return pl.pallas_call(
        _tclayer_body,
        grid=(_NB,),
        in_specs=[
            pl.BlockSpec((_BR, H), lambda i: (i, 0)),
            pl.BlockSpec((_BR, H), lambda i: (i, 0)),
            pl.BlockSpec((_BR, 1), lambda i: (i, 0)),
            pl.BlockSpec((_BR, 1), lambda i: (i, 0)),
            pl.BlockSpec((H, H), lambda i: (0, 0)),
            pl.BlockSpec((1, H), lambda i: (0, 0)),
            pl.BlockSpec((H, 1), lambda i: (0, 0)),
            pl.BlockSpec((H, 1), lambda i: (0, 0)),
        ],
        out_specs=[
            pl.BlockSpec((_BR, H), lambda i: (i, 0)),
            pl.BlockSpec((_BR, 1), lambda i: (i, 0)),
            pl.BlockSpec((_BR, 1), lambda i: (i, 0)),
        ],
        out_shape=[
            jax.ShapeDtypeStruct((N, H), jnp.float32),
            jax.ShapeDtypeStruct((N, 1), jnp.float32),
            jax.ShapeDtypeStruct((N, 1), jnp.float32),
        ],
    )(a0, a1, s0, s1, wt, b, asv, adv)


def _tcfinal_body(h_ref, w1h_ref, c1_ref, wns2_ref, bns2_ref, wst_ref, wdt_ref,
                  nsc_ref, ps_ref, pd_ref):
    h = h_ref[...]
    nsh = jnp.maximum(
        jnp.dot(h, w1h_ref[...], preferred_element_type=jnp.float32)
        + c1_ref[...], 0.0)
    logit = jnp.dot(nsh, wns2_ref[...],
                    preferred_element_type=jnp.float32) + bns2_ref[...]
    nsc_ref[...] = 1.0 / (1.0 + jnp.exp(-logit))
    ps_ref[...] = jnp.dot(h, wst_ref[...], preferred_element_type=jnp.float32)
    pd_ref[...] = jnp.dot(h, wdt_ref[...], preferred_element_type=jnp.float32)


def _tcfinal(h, w1h, c1, wns2, bns2, wst, wdt):
    return pl.pallas_call(
        _tcfinal_body,
        grid=(_NB,),
        in_specs=[
            pl.BlockSpec((_BR, H), lambda i: (i, 0)),
            pl.BlockSpec((H, H), lambda i: (0, 0)),
            pl.BlockSpec((1, H), lambda i: (0, 0)),
            pl.BlockSpec((H, 1), lambda i: (0, 0)),
            pl.BlockSpec((1, 1), lambda i: (0, 0)),
            pl.BlockSpec((H, H), lambda i: (0, 0)),
            pl.BlockSpec((H, H), lambda i: (0, 0)),
        ],
        out_specs=[
            pl.BlockSpec((_BR, 1), lambda i: (i, 0)),
            pl.BlockSpec((_BR, H), lambda i: (i, 0)),
            pl.BlockSpec((_BR, H), lambda i: (i, 0)),
        ],
        out_shape=[
            jax.ShapeDtypeStruct((N, 1), jnp.float32),
            jax.ShapeDtypeStruct((N, H), jnp.float32),
            jax.ShapeDtypeStruct((N, H), jnp.float32),
        ],
    )(h, w1h, c1, wns2, bns2, wst, wdt)


def _tcprep_body(q_ref, wqt_ref, bq_ref, ed_ref, wst_ref, bs_ref, at_ref,
                 wtes_ref, wqes_ref, bes_ref, w1qt_ref, bns1_ref,
                 qh_ref, te_ref, att_ref, ct_ref, c1_ref):
    qh = jnp.dot(q_ref[...], wqt_ref[...],
                 preferred_element_type=jnp.float32) + bq_ref[...]
    qh_ref[...] = qh
    te = jnp.dot(ed_ref[...], wst_ref[...],
                 preferred_element_type=jnp.float32) + bs_ref[...]
    te_ref[...] = te
    att_ref[...] = jax.lax.dot_general(
        at_ref[...], te, (((1,), (1,)), ((), ())),
        preferred_element_type=jnp.float32)
    ct_ref[...] = (jnp.dot(te, wtes_ref[...], preferred_element_type=jnp.float32)
                   + jnp.dot(qh, wqes_ref[...],
                             preferred_element_type=jnp.float32)
                   + bes_ref[...])
    c1_ref[...] = jnp.dot(qh, w1qt_ref[...],
                          preferred_element_type=jnp.float32) + bns1_ref[...]


def _tcprep(q2, wqt, bq, ed, wst, bs, at, wtes, wqes, bes, w1qt, bns1):
    return pl.pallas_call(
        _tcprep_body,
        out_shape=[
            jax.ShapeDtypeStruct((1, H), jnp.float32),
            jax.ShapeDtypeStruct((NT, 16), jnp.float32),
            jax.ShapeDtypeStruct((L, NT), jnp.float32),
            jax.ShapeDtypeStruct((NT, H), jnp.float32),
            jax.ShapeDtypeStruct((1, H), jnp.float32),
        ],
    )(q2, wqt, bq, ed, wst, bs, at, wtes, wqes, bes, w1qt, bns1)


@jax.jit
def kernel(node_features, edge_index, edge_type, edge_descriptor, query,
           W_node_in, b_node_in, W_query_in, b_query_in, W_schema, b_schema,
           a_src, a_dst, a_type, W_mp, b_mp,
           W_ns1, b_ns1, W_ns2, b_ns2, W_es1, b_es1, W_es2, b_es2):
    src = edge_index[0].astype(jnp.int32)
    dst = edge_index[1].astype(jnp.int32)
    et = edge_type.astype(jnp.int32)
    pad = EP - E
    zpad = jnp.zeros((pad,), jnp.int32)
    src2 = jnp.concatenate([src, zpad]).reshape(EP // C, C)
    et2 = jnp.concatenate([et, zpad]).reshape(EP // C, C)
    dst2s = jnp.concatenate([dst, jnp.full((pad,), N, jnp.int32)]).reshape(
        EP // C, C)
    dst2g = jnp.concatenate([dst, zpad]).reshape(EP // C, C)

    qh, type_emb, att_all, ct, c1 = _tcprep(
        query.reshape(1, H), W_query_in.T, b_query_in.reshape(1, H),
        edge_descriptor, W_schema.T, b_schema.reshape(1, 16),
        a_type, W_es1[:, 2 * H:2 * H + NT].T, W_es1[:, 2 * H + NT:].T,
        b_es1.reshape(1, H), W_ns1[:, H:].T, b_ns1.reshape(1, H))
    att_pad = jnp.pad(att_all, ((0, 0), (0, 128 - NT)))

    h, asn, adn = _tc0(node_features, W_node_in.T, b_node_in.reshape(1, H),
                       a_src[0].reshape(H, 1), a_dst[0].reshape(H, 1))

    npad = jnp.zeros((NP2 - N,), jnp.float32)
    for l in range(L):
        asn_p = jnp.concatenate([asn.reshape(N), npad])
        adn_p = jnp.concatenate([adn.reshape(N), npad])
        s_p, agg_p = _layer_sc(src2, dst2s, et2, h, asn_p, adn_p, att_pad[l])
        nl = min(l + 1, L - 1)
        h, asn, adn = _tclayer(agg_p[0], agg_p[1],
                               s_p[0, :N].reshape(N, 1),
                               s_p[1, :N].reshape(N, 1),
                               W_mp[l].T, b_mp[l].reshape(1, H),
                               a_src[nl].reshape(H, 1), a_dst[nl].reshape(H, 1))

    nscore, ps, pd = _tcfinal(h, W_ns1[:, :H].T, c1, W_ns2.T,
                              b_ns2.reshape(1, 1),
                              W_es1[:, :H].T, W_es1[:, H:2 * H].T)

    b2v = jnp.full((128,), b_es2[0], jnp.float32)
    (escore,) = _edge_sc(src2, dst2g, et2, ps, pd, ct, W_es2[0], b2v)

    return nscore.reshape(N), escore[:E], h, type_emb


# speedup vs baseline: 1.1971x; 1.1971x over previous
import jax
import jax.numpy as jnp
from jax import lax
from jax.experimental import pallas as pl
from jax.experimental.pallas import tpu as pltpu
from jax.experimental.pallas import tpu_sc as plsc

N = 10000
E = 320000
H = 128
NT = 16
L = 3

NC = 2
NS = 16
NW = NC * NS
C = 128
NCH = 80
EPW = NCH * C
EP = NW * EPW
NP2 = 10240
NPAD = N + 16

_MESH = plsc.VectorSubcoreMesh(
    core_axis_name="c", subcore_axis_name="s", num_cores=NC, num_subcores=NS)


GRP = 16
NGRP = NCH // GRP
LCH = (112, 48)
ECH = (96, 64)


def _layer_sc_body(src_hbm, dst_hbm, et_hbm, h_hbm, asn_hbm, adn_hbm, att_hbm,
                   s_out, agg_out,
                   src_g, dst_g, et_g, att_v, asn0, adn0, asn1, adn1,
                   ex0, ex1, row0, row1,
                   zbuf, zvec, shared_s, shared_agg, sem1, sem2):
    c = lax.axis_index("c")
    s = lax.axis_index("s")
    nch_c = jnp.where(c == 0, LCH[0], LCH[1])
    base = jnp.where(c == 0, s * LCH[0], NS * LCH[0] + s * LCH[1])

    for i in range(8):
        for k in range(8):
            zbuf[i, pl.ds(k * 16, 16)] = jnp.zeros((16,), jnp.float32)
    for i in range(40):
        zvec[pl.ds(i * 16, 16)] = jnp.zeros((16,), jnp.float32)

    def zagg(i, _):
        pltpu.sync_copy(zbuf, shared_agg.at[pl.ds(s * 640 + i * 8, 8), :])
        return _
    lax.fori_loop(0, 80, zagg, 0)
    pltpu.sync_copy(zvec, shared_s.at[pl.ds(s * 640, 640)])

    pltpu.sync_copy(att_hbm, att_v)

    plsc.subcore_barrier()

    def fire(jj, row, asn_c, adn_c, sem):
        pltpu.async_copy(h_hbm.at[src_g.at[jj]], row, sem)
        pltpu.async_copy(asn_hbm.at[src_g.at[jj]], asn_c, sem)
        pltpu.async_copy(adn_hbm.at[dst_g.at[jj]], adn_c, sem)

    def drain(row, asn_c, adn_c, sem):
        pltpu.make_async_copy(h_hbm.at[pl.ds(0, C)], row, sem).wait()
        pltpu.make_async_copy(asn_hbm.at[pl.ds(0, C)], asn_c, sem).wait()
        pltpu.make_async_copy(adn_hbm.at[pl.ds(0, C)], adn_c, sem).wait()

    def compute(jj, row, asn_c, adn_c, exv_b):
        for gg in range(C // 16):
            ti = et_g[jj, pl.ds(gg * 16, 16)]
            z = (asn_c[pl.ds(gg * 16, 16)] + adn_c[pl.ds(gg * 16, 16)]
                 + plsc.load_gather(att_v, [ti]))
            e = jnp.where(z >= 0.0, z, z * jnp.float32(0.2))
            exv_b[pl.ds(gg * 16, 16)] = jnp.exp(e)

        def gbody(gg, _):
            exvec = exv_b[pl.ds(gg * 16, 16)]
            for rr in range(16):
                v = jnp.full((16,), exvec[rr], jnp.float32)
                r = gg * 16 + rr
                for k in range(H // 16):
                    row[r, pl.ds(k * 16, 16)] = row[r, pl.ds(k * 16, 16)] * v
            return _
        lax.fori_loop(0, C // 16, gbody, 0)

        idxd = dst_g.at[jj]
        pltpu.sync_copy(row, shared_agg.at[idxd], add=True)
        pltpu.sync_copy(exv_b, shared_s.at[idxd], add=True)

    def group(g, _):
        pltpu.sync_copy(src_hbm.at[pl.ds(base + g * GRP, GRP), :], src_g)
        pltpu.sync_copy(dst_hbm.at[pl.ds(base + g * GRP, GRP), :], dst_g)
        pltpu.sync_copy(et_hbm.at[pl.ds(base + g * GRP, GRP), :], et_g)

        fire(0, row0, asn0, adn0, sem1)

        def pair(p, carry):
            fire(2 * p + 1, row1, asn1, adn1, sem2)
            drain(row0, asn0, adn0, sem1)
            compute(2 * p, row0, asn0, adn0, ex0)

            @pl.when(p < GRP // 2 - 1)
            def _fire_next():
                fire(2 * p + 2, row0, asn0, adn0, sem1)
            drain(row1, asn1, adn1, sem2)
            compute(2 * p + 1, row1, asn1, adn1, ex1)
            return carry
        lax.fori_loop(0, GRP // 2, pair, 0)
        return _
    lax.fori_loop(0, nch_c // GRP, group, 0)

    plsc.subcore_barrier()

    @pl.when(s < 15)
    def _():
        pltpu.sync_copy(shared_agg.at[pl.ds(s * 624, 624), :],
                        agg_out.at[c, pl.ds(s * 624, 624), :])

    @pl.when(s == 15)
    def _():
        pltpu.sync_copy(shared_agg.at[pl.ds(15 * 624, 640), :],
                        agg_out.at[c, pl.ds(15 * 624, 640), :])

    @pl.when(s == 0)
    def _():
        pltpu.sync_copy(shared_s, s_out.at[c])


_layer_sc = pl.kernel(
    _layer_sc_body,
    out_type=[
        jax.ShapeDtypeStruct((NC, NP2), jnp.float32),
        jax.ShapeDtypeStruct((NC, N, H), jnp.float32),
    ],
    mesh=_MESH,
    compiler_params=pltpu.CompilerParams(needs_layout_passes=False),
    scratch_types=[
        pltpu.VMEM((GRP, C), jnp.int32),
        pltpu.VMEM((GRP, C), jnp.int32),
        pltpu.VMEM((GRP, C), jnp.int32),
        pltpu.VMEM((128,), jnp.float32),
        pltpu.VMEM((C,), jnp.float32),
        pltpu.VMEM((C,), jnp.float32),
        pltpu.VMEM((C,), jnp.float32),
        pltpu.VMEM((C,), jnp.float32),
        pltpu.VMEM((C,), jnp.float32),
        pltpu.VMEM((C,), jnp.float32),
        pltpu.VMEM((C, H), jnp.float32),
        pltpu.VMEM((C, H), jnp.float32),
        pltpu.VMEM((8, H), jnp.float32),
        pltpu.VMEM((640,), jnp.float32),
        pltpu.VMEM_SHARED((NP2,), jnp.float32),
        pltpu.VMEM_SHARED((NP2, H), jnp.float32),
        pltpu.SemaphoreType.DMA,
        pltpu.SemaphoreType.DMA,
    ],
)


def _edge_sc_body(src_hbm, dst_hbm, et_hbm, ps_hbm, pd_hbm, ct_hbm, w2_hbm,
                  b2_hbm, out_hbm,
                  srcv, dstv, etv, ctv, w2v, b2v, ps0, pd0, ps1, pd1,
                  outv, sem1, sem2):
    c = lax.axis_index("c")
    s = lax.axis_index("s")
    nch_c = jnp.where(c == 0, ECH[0], ECH[1])
    base = jnp.where(c == 0, s * ECH[0], NS * ECH[0] + s * ECH[1])

    @pl.when(c == 0)
    def _():
        pltpu.sync_copy(src_hbm.at[pl.ds(base, ECH[0]), :],
                        srcv.at[pl.ds(0, ECH[0])])
        pltpu.sync_copy(dst_hbm.at[pl.ds(base, ECH[0]), :],
                        dstv.at[pl.ds(0, ECH[0])])
        pltpu.sync_copy(et_hbm.at[pl.ds(base, ECH[0]), :],
                        etv.at[pl.ds(0, ECH[0])])

    @pl.when(c == 1)
    def _():
        pltpu.sync_copy(src_hbm.at[pl.ds(base, ECH[1]), :],
                        srcv.at[pl.ds(0, ECH[1])])
        pltpu.sync_copy(dst_hbm.at[pl.ds(base, ECH[1]), :],
                        dstv.at[pl.ds(0, ECH[1])])
        pltpu.sync_copy(et_hbm.at[pl.ds(base, ECH[1]), :],
                        etv.at[pl.ds(0, ECH[1])])
    pltpu.sync_copy(ct_hbm, ctv)
    pltpu.sync_copy(w2_hbm, w2v)
    pltpu.sync_copy(b2_hbm, b2v)

    w2s = [w2v[pl.ds(k * 16, 16)] for k in range(H // 16)]
    b2vec = b2v[pl.ds(0, 16)]
    lanes = lax.broadcasted_iota(jnp.int32, (16,), 0)

    def fire(j, psb, pdb, sem):
        pltpu.async_copy(ps_hbm.at[srcv.at[j]], psb, sem)
        pltpu.async_copy(pd_hbm.at[dstv.at[j]], pdb, sem)

    def drain(psb, pdb, sem):
        pltpu.make_async_copy(ps_hbm.at[pl.ds(0, C), :], psb, sem).wait()
        pltpu.make_async_copy(pd_hbm.at[pl.ds(0, C), :], pdb, sem).wait()

    def compute(j, psb, pdb):
        def gbody(g, _):
            etvec = etv[j, pl.ds(g * 16, 16)]
            lvec = jnp.zeros((16,), jnp.float32)
            for rr in range(16):
                et_r = etvec[rr]
                r = g * 16 + rr
                acc = jnp.zeros((16,), jnp.float32)
                for k in range(H // 16):
                    t = (psb[r, pl.ds(k * 16, 16)]
                         + pdb[r, pl.ds(k * 16, 16)]
                         + ctv[et_r, pl.ds(k * 16, 16)])
                    t = jnp.maximum(t, 0.0)
                    acc = acc + t * w2s[k]
                lvec = jnp.where(lanes == rr, jnp.sum(acc), lvec)
            v = lvec + b2vec
            outv[pl.ds(j * C + g * 16, 16)] = 1.0 / (1.0 + jnp.exp(-v))
            return _
        lax.fori_loop(0, C // 16, gbody, 0)

    fire(0, ps0, pd0, sem1)

    def pair(p, carry):
        fire(2 * p + 1, ps1, pd1, sem2)
        drain(ps0, pd0, sem1)
        compute(2 * p, ps0, pd0)

        @pl.when(p < nch_c // 2 - 1)
        def _fire_next():
            fire(2 * p + 2, ps0, pd0, sem1)
        drain(ps1, pd1, sem2)
        compute(2 * p + 1, ps1, pd1)
        return carry
    lax.fori_loop(0, nch_c // 2, pair, 0)

    @pl.when(c == 0)
    def _():
        pltpu.sync_copy(outv.at[pl.ds(0, ECH[0] * C)],
                        out_hbm.at[pl.ds(base * C, ECH[0] * C)])

    @pl.when(c == 1)
    def _():
        pltpu.sync_copy(outv.at[pl.ds(0, ECH[1] * C)],
                        out_hbm.at[pl.ds(base * C, ECH[1] * C)])


_edge_sc = pl.kernel(
    _edge_sc_body,
    out_type=[jax.ShapeDtypeStruct((EP,), jnp.float32)],
    mesh=_MESH,
    compiler_params=pltpu.CompilerParams(needs_layout_passes=False),
    scratch_types=[
        pltpu.VMEM((max(ECH), C), jnp.int32),
        pltpu.VMEM((max(ECH), C), jnp.int32),
        pltpu.VMEM((max(ECH), C), jnp.int32),
        pltpu.VMEM((NT, H), jnp.float32),
        pltpu.VMEM((H,), jnp.float32),
        pltpu.VMEM((128,), jnp.float32),
        pltpu.VMEM((C, H), jnp.float32),
        pltpu.VMEM((C, H), jnp.float32),
        pltpu.VMEM((C, H), jnp.float32),
        pltpu.VMEM((C, H), jnp.float32),
        pltpu.VMEM((max(ECH) * C,), jnp.float32),
        pltpu.SemaphoreType.DMA,
        pltpu.SemaphoreType.DMA,
    ],
)


_BR = 1000
_NB = N // _BR


def _tc0_body(x_ref, wt_ref, b_ref, asv_ref, adv_ref, h_ref, asn_ref, adn_ref):
    h = jnp.dot(x_ref[...], wt_ref[...],
                preferred_element_type=jnp.float32) + b_ref[...]
    h_ref[...] = h
    asn_ref[...] = jnp.dot(h, asv_ref[...], preferred_element_type=jnp.float32)
    adn_ref[...] = jnp.dot(h, adv_ref[...], preferred_element_type=jnp.float32)


def _tc0(x, wt, b, asv, adv):
    return pl.pallas_call(
        _tc0_body,
        grid=(_NB,),
        in_specs=[
            pl.BlockSpec((_BR, H), lambda i: (i, 0)),
            pl.BlockSpec((H, H), lambda i: (0, 0)),
            pl.BlockSpec((1, H), lambda i: (0, 0)),
            pl.BlockSpec((H, 1), lambda i: (0, 0)),
            pl.BlockSpec((H, 1), lambda i: (0, 0)),
        ],
        out_specs=[
            pl.BlockSpec((_BR, H), lambda i: (i, 0)),
            pl.BlockSpec((_BR, 1), lambda i: (i, 0)),
            pl.BlockSpec((_BR, 1), lambda i: (i, 0)),
        ],
        out_shape=[
            jax.ShapeDtypeStruct((N, H), jnp.float32),
            jax.ShapeDtypeStruct((N, 1), jnp.float32),
            jax.ShapeDtypeStruct((N, 1), jnp.float32),
        ],
    )(x, wt, b, asv, adv)


def _tclayer_body(a0_ref, a1_ref, s0_ref, s1_ref, wt_ref, b_ref, asv_ref,
                  adv_ref, h_ref, asn_ref, adn_ref):
    inv = 1.0 / (s0_ref[...] + s1_ref[...] + jnp.float32(1e-16))
    x = (a0_ref[...] + a1_ref[...]) * inv
    h = jnp.dot(x, wt_ref[...], preferred_element_type=jnp.float32) + b_ref[...]
    h = jnp.maximum(h, 0.0)
    h_ref[...] = h
    asn_ref[...] = jnp.dot(h, asv_ref[...], preferred_element_type=jnp.float32)
    adn_ref[...] = jnp.dot(h, adv_ref[...], preferred_element_type=jnp.float32)


def _tclayer(a0, a1, s0, s1, wt, b, asv, adv):
    return pl.pallas_call(
        _tclayer_body,
        grid=(_NB,),
        in_specs=[
            pl.BlockSpec((_BR, H), lambda i: (i, 0)),
            pl.BlockSpec((_BR, H), lambda i: (i, 0)),
            pl.BlockSpec((_BR, 1), lambda i: (i, 0)),
            pl.BlockSpec((_BR, 1), lambda i: (i, 0)),
            pl.BlockSpec((H, H), lambda i: (0, 0)),
            pl.BlockSpec((1, H), lambda i: (0, 0)),
            pl.BlockSpec((H, 1), lambda i: (0, 0)),
            pl.BlockSpec((H, 1), lambda i: (0, 0)),
        ],
        out_specs=[
            pl.BlockSpec((_BR, H), lambda i: (i, 0)),
            pl.BlockSpec((_BR, 1), lambda i: (i, 0)),
            pl.BlockSpec((_BR, 1), lambda i: (i, 0)),
        ],
        out_shape=[
            jax.ShapeDtypeStruct((N, H), jnp.float32),
            jax.ShapeDtypeStruct((N, 1), jnp.float32),
            jax.ShapeDtypeStruct((N, 1), jnp.float32),
        ],
    )(a0, a1, s0, s1, wt, b, asv, adv)


def _tcfinal_body(h_ref, w1h_ref, c1_ref, wns2_ref, bns2_ref, wst_ref, wdt_ref,
                  nsc_ref, ps_ref, pd_ref):
    h = h_ref[...]
    nsh = jnp.maximum(
        jnp.dot(h, w1h_ref[...], preferred_element_type=jnp.float32)
        + c1_ref[...], 0.0)
    logit = jnp.dot(nsh, wns2_ref[...],
                    preferred_element_type=jnp.float32) + bns2_ref[...]
    nsc_ref[...] = 1.0 / (1.0 + jnp.exp(-logit))
    ps_ref[...] = jnp.dot(h, wst_ref[...], preferred_element_type=jnp.float32)
    pd_ref[...] = jnp.dot(h, wdt_ref[...], preferred_element_type=jnp.float32)


def _tcfinal(h, w1h, c1, wns2, bns2, wst, wdt):
    return pl.pallas_call(
        _tcfinal_body,
        grid=(_NB,),
        in_specs=[
            pl.BlockSpec((_BR, H), lambda i: (i, 0)),
            pl.BlockSpec((H, H), lambda i: (0, 0)),
            pl.BlockSpec((1, H), lambda i: (0, 0)),
            pl.BlockSpec((H, 1), lambda i: (0, 0)),
            pl.BlockSpec((1, 1), lambda i: (0, 0)),
            pl.BlockSpec((H, H), lambda i: (0, 0)),
            pl.BlockSpec((H, H), lambda i: (0, 0)),
        ],
        out_specs=[
            pl.BlockSpec((_BR, 1), lambda i: (i, 0)),
            pl.BlockSpec((_BR, H), lambda i: (i, 0)),
            pl.BlockSpec((_BR, H), lambda i: (i, 0)),
        ],
        out_shape=[
            jax.ShapeDtypeStruct((N, 1), jnp.float32),
            jax.ShapeDtypeStruct((N, H), jnp.float32),
            jax.ShapeDtypeStruct((N, H), jnp.float32),
        ],
    )(h, w1h, c1, wns2, bns2, wst, wdt)


def _tcprep_body(q_ref, wqt_ref, bq_ref, ed_ref, wst_ref, bs_ref, at_ref,
                 wtes_ref, wqes_ref, bes_ref, w1qt_ref, bns1_ref,
                 qh_ref, te_ref, att_ref, ct_ref, c1_ref):
    qh = jnp.dot(q_ref[...], wqt_ref[...],
                 preferred_element_type=jnp.float32) + bq_ref[...]
    qh_ref[...] = qh
    te = jnp.dot(ed_ref[...], wst_ref[...],
                 preferred_element_type=jnp.float32) + bs_ref[...]
    te_ref[...] = te
    att_ref[...] = jax.lax.dot_general(
        at_ref[...], te, (((1,), (1,)), ((), ())),
        preferred_element_type=jnp.float32)
    ct_ref[...] = (jnp.dot(te, wtes_ref[...], preferred_element_type=jnp.float32)
                   + jnp.dot(qh, wqes_ref[...],
                             preferred_element_type=jnp.float32)
                   + bes_ref[...])
    c1_ref[...] = jnp.dot(qh, w1qt_ref[...],
                          preferred_element_type=jnp.float32) + bns1_ref[...]


def _tcprep(q2, wqt, bq, ed, wst, bs, at, wtes, wqes, bes, w1qt, bns1):
    return pl.pallas_call(
        _tcprep_body,
        out_shape=[
            jax.ShapeDtypeStruct((1, H), jnp.float32),
            jax.ShapeDtypeStruct((NT, 16), jnp.float32),
            jax.ShapeDtypeStruct((L, NT), jnp.float32),
            jax.ShapeDtypeStruct((NT, H), jnp.float32),
            jax.ShapeDtypeStruct((1, H), jnp.float32),
        ],
    )(q2, wqt, bq, ed, wst, bs, at, wtes, wqes, bes, w1qt, bns1)


@jax.jit
def kernel(node_features, edge_index, edge_type, edge_descriptor, query,
           W_node_in, b_node_in, W_query_in, b_query_in, W_schema, b_schema,
           a_src, a_dst, a_type, W_mp, b_mp,
           W_ns1, b_ns1, W_ns2, b_ns2, W_es1, b_es1, W_es2, b_es2):
    src = edge_index[0].astype(jnp.int32)
    dst = edge_index[1].astype(jnp.int32)
    et = edge_type.astype(jnp.int32)
    pad = EP - E
    zpad = jnp.zeros((pad,), jnp.int32)
    src2 = jnp.concatenate([src, zpad]).reshape(EP // C, C)
    et2 = jnp.concatenate([et, zpad]).reshape(EP // C, C)
    dst2s = jnp.concatenate([dst, jnp.full((pad,), N, jnp.int32)]).reshape(
        EP // C, C)
    dst2g = jnp.concatenate([dst, zpad]).reshape(EP // C, C)

    qh, type_emb, att_all, ct, c1 = _tcprep(
        query.reshape(1, H), W_query_in.T, b_query_in.reshape(1, H),
        edge_descriptor, W_schema.T, b_schema.reshape(1, 16),
        a_type, W_es1[:, 2 * H:2 * H + NT].T, W_es1[:, 2 * H + NT:].T,
        b_es1.reshape(1, H), W_ns1[:, H:].T, b_ns1.reshape(1, H))
    att_pad = jnp.pad(att_all, ((0, 0), (0, 128 - NT)))

    h, asn, adn = _tc0(node_features, W_node_in.T, b_node_in.reshape(1, H),
                       a_src[0].reshape(H, 1), a_dst[0].reshape(H, 1))

    npad = jnp.zeros((NP2 - N,), jnp.float32)
    for l in range(L):
        asn_p = jnp.concatenate([asn.reshape(N), npad])
        adn_p = jnp.concatenate([adn.reshape(N), npad])
        s_p, agg_p = _layer_sc(src2, dst2s, et2, h, asn_p, adn_p, att_pad[l])
        nl = min(l + 1, L - 1)
        h, asn, adn = _tclayer(agg_p[0], agg_p[1],
                               s_p[0, :N].reshape(N, 1),
                               s_p[1, :N].reshape(N, 1),
                               W_mp[l].T, b_mp[l].reshape(1, H),
                               a_src[nl].reshape(H, 1), a_dst[nl].reshape(H, 1))

    nscore, ps, pd = _tcfinal(h, W_ns1[:, :H].T, c1, W_ns2.T,
                              b_ns2.reshape(1, 1),
                              W_es1[:, :H].T, W_es1[:, H:2 * H].T)

    b2v = jnp.full((128,), b_es2[0], jnp.float32)
    (escore,) = _edge_sc(src2, dst2g, et2, ps, pd, ct, W_es2[0], b2v)

    return nscore.reshape(N), escore[:E], h, type_emb


# speedup vs baseline: 1.2168x; 1.0164x over previous
import jax
import jax.numpy as jnp
from jax import lax
from jax.experimental import pallas as pl
from jax.experimental.pallas import tpu as pltpu
from jax.experimental.pallas import tpu_sc as plsc

N = 10000
E = 320000
H = 128
NT = 16
L = 3

NC = 2
NS = 16
NW = NC * NS
C = 128
NCH = 80
EPW = NCH * C
EP = NW * EPW
NP2 = 10240
NPAD = N + 16

_MESH = plsc.VectorSubcoreMesh(
    core_axis_name="c", subcore_axis_name="s", num_cores=NC, num_subcores=NS)


GRP = 16
NGRP = NCH // GRP
LCH = (128, 32)
ECH = (96, 64)


def _layer_sc_body(src_hbm, dst_hbm, et_hbm, h_hbm, asn_hbm, adn_hbm, att_hbm,
                   s_out, agg_out,
                   src_g, dst_g, et_g, att_v, asn0, adn0, asn1, adn1,
                   ex0, ex1, row0, row1,
                   zbuf, zvec, shared_s, shared_agg, sem1, sem2):
    c = lax.axis_index("c")
    s = lax.axis_index("s")
    nch_c = jnp.where(c == 0, LCH[0], LCH[1])
    base = jnp.where(c == 0, s * LCH[0], NS * LCH[0] + s * LCH[1])

    for i in range(8):
        for k in range(8):
            zbuf[i, pl.ds(k * 16, 16)] = jnp.zeros((16,), jnp.float32)
    for i in range(40):
        zvec[pl.ds(i * 16, 16)] = jnp.zeros((16,), jnp.float32)

    def zagg(i, _):
        pltpu.sync_copy(zbuf, shared_agg.at[pl.ds(s * 640 + i * 8, 8), :])
        return _
    lax.fori_loop(0, 80, zagg, 0)
    pltpu.sync_copy(zvec, shared_s.at[pl.ds(s * 640, 640)])

    pltpu.sync_copy(att_hbm, att_v)

    plsc.subcore_barrier()

    def fire(jj, row, asn_c, adn_c, sem):
        pltpu.async_copy(h_hbm.at[src_g.at[jj]], row, sem)
        pltpu.async_copy(asn_hbm.at[src_g.at[jj]], asn_c, sem)
        pltpu.async_copy(adn_hbm.at[dst_g.at[jj]], adn_c, sem)

    def drain(row, asn_c, adn_c, sem):
        pltpu.make_async_copy(h_hbm.at[pl.ds(0, C)], row, sem).wait()
        pltpu.make_async_copy(asn_hbm.at[pl.ds(0, C)], asn_c, sem).wait()
        pltpu.make_async_copy(adn_hbm.at[pl.ds(0, C)], adn_c, sem).wait()

    def compute(jj, row, asn_c, adn_c, exv_b):
        for gg in range(C // 16):
            ti = et_g[jj, pl.ds(gg * 16, 16)]
            z = (asn_c[pl.ds(gg * 16, 16)] + adn_c[pl.ds(gg * 16, 16)]
                 + plsc.load_gather(att_v, [ti]))
            e = jnp.where(z >= 0.0, z, z * jnp.float32(0.2))
            exv_b[pl.ds(gg * 16, 16)] = jnp.exp(e)

        def gbody(gg, _):
            exvec = exv_b[pl.ds(gg * 16, 16)]
            for rr in range(16):
                v = jnp.full((16,), exvec[rr], jnp.float32)
                r = gg * 16 + rr
                for k in range(H // 16):
                    row[r, pl.ds(k * 16, 16)] = row[r, pl.ds(k * 16, 16)] * v
            return _
        lax.fori_loop(0, C // 16, gbody, 0)

        idxd = dst_g.at[jj]
        pltpu.sync_copy(row, shared_agg.at[idxd], add=True)
        pltpu.sync_copy(exv_b, shared_s.at[idxd], add=True)

    def group(g, _):
        pltpu.sync_copy(src_hbm.at[pl.ds(base + g * GRP, GRP), :], src_g)
        pltpu.sync_copy(dst_hbm.at[pl.ds(base + g * GRP, GRP), :], dst_g)
        pltpu.sync_copy(et_hbm.at[pl.ds(base + g * GRP, GRP), :], et_g)

        fire(0, row0, asn0, adn0, sem1)

        def pair(p, carry):
            fire(2 * p + 1, row1, asn1, adn1, sem2)
            drain(row0, asn0, adn0, sem1)
            compute(2 * p, row0, asn0, adn0, ex0)

            @pl.when(p < GRP // 2 - 1)
            def _fire_next():
                fire(2 * p + 2, row0, asn0, adn0, sem1)
            drain(row1, asn1, adn1, sem2)
            compute(2 * p + 1, row1, asn1, adn1, ex1)
            return carry
        lax.fori_loop(0, GRP // 2, pair, 0)
        return _
    lax.fori_loop(0, nch_c // GRP, group, 0)

    plsc.subcore_barrier()

    @pl.when(s < 15)
    def _():
        pltpu.sync_copy(shared_agg.at[pl.ds(s * 624, 624), :],
                        agg_out.at[c, pl.ds(s * 624, 624), :])

    @pl.when(s == 15)
    def _():
        pltpu.sync_copy(shared_agg.at[pl.ds(15 * 624, 640), :],
                        agg_out.at[c, pl.ds(15 * 624, 640), :])

    @pl.when(s == 0)
    def _():
        pltpu.sync_copy(shared_s, s_out.at[c])


_layer_sc = pl.kernel(
    _layer_sc_body,
    out_type=[
        jax.ShapeDtypeStruct((NC, NP2), jnp.float32),
        jax.ShapeDtypeStruct((NC, N, H), jnp.float32),
    ],
    mesh=_MESH,
    compiler_params=pltpu.CompilerParams(needs_layout_passes=False),
    scratch_types=[
        pltpu.VMEM((GRP, C), jnp.int32),
        pltpu.VMEM((GRP, C), jnp.int32),
        pltpu.VMEM((GRP, C), jnp.int32),
        pltpu.VMEM((128,), jnp.float32),
        pltpu.VMEM((C,), jnp.float32),
        pltpu.VMEM((C,), jnp.float32),
        pltpu.VMEM((C,), jnp.float32),
        pltpu.VMEM((C,), jnp.float32),
        pltpu.VMEM((C,), jnp.float32),
        pltpu.VMEM((C,), jnp.float32),
        pltpu.VMEM((C, H), jnp.float32),
        pltpu.VMEM((C, H), jnp.float32),
        pltpu.VMEM((8, H), jnp.float32),
        pltpu.VMEM((640,), jnp.float32),
        pltpu.VMEM_SHARED((NP2,), jnp.float32),
        pltpu.VMEM_SHARED((NP2, H), jnp.float32),
        pltpu.SemaphoreType.DMA,
        pltpu.SemaphoreType.DMA,
    ],
)


def _edge_sc_body(src_hbm, dst_hbm, et_hbm, ps_hbm, pd_hbm, ct_hbm, w2_hbm,
                  b2_hbm, out_hbm,
                  srcv, dstv, etv, ctv, w2v, b2v, ps0, pd0, ps1, pd1,
                  outv, sem1, sem2):
    c = lax.axis_index("c")
    s = lax.axis_index("s")
    nch_c = jnp.where(c == 0, ECH[0], ECH[1])
    base = jnp.where(c == 0, s * ECH[0], NS * ECH[0] + s * ECH[1])

    @pl.when(c == 0)
    def _():
        pltpu.sync_copy(src_hbm.at[pl.ds(base, ECH[0]), :],
                        srcv.at[pl.ds(0, ECH[0])])
        pltpu.sync_copy(dst_hbm.at[pl.ds(base, ECH[0]), :],
                        dstv.at[pl.ds(0, ECH[0])])
        pltpu.sync_copy(et_hbm.at[pl.ds(base, ECH[0]), :],
                        etv.at[pl.ds(0, ECH[0])])

    @pl.when(c == 1)
    def _():
        pltpu.sync_copy(src_hbm.at[pl.ds(base, ECH[1]), :],
                        srcv.at[pl.ds(0, ECH[1])])
        pltpu.sync_copy(dst_hbm.at[pl.ds(base, ECH[1]), :],
                        dstv.at[pl.ds(0, ECH[1])])
        pltpu.sync_copy(et_hbm.at[pl.ds(base, ECH[1]), :],
                        etv.at[pl.ds(0, ECH[1])])
    pltpu.sync_copy(ct_hbm, ctv)
    pltpu.sync_copy(w2_hbm, w2v)
    pltpu.sync_copy(b2_hbm, b2v)

    w2s = [w2v[pl.ds(k * 16, 16)] for k in range(H // 16)]
    b2vec = b2v[pl.ds(0, 16)]
    lanes = lax.broadcasted_iota(jnp.int32, (16,), 0)

    def fire(j, psb, pdb, sem):
        pltpu.async_copy(ps_hbm.at[srcv.at[j]], psb, sem)
        pltpu.async_copy(pd_hbm.at[dstv.at[j]], pdb, sem)

    def drain(psb, pdb, sem):
        pltpu.make_async_copy(ps_hbm.at[pl.ds(0, C), :], psb, sem).wait()
        pltpu.make_async_copy(pd_hbm.at[pl.ds(0, C), :], pdb, sem).wait()

    def compute(j, psb, pdb):
        def gbody(g, _):
            etvec = etv[j, pl.ds(g * 16, 16)]
            lvec = jnp.zeros((16,), jnp.float32)
            for rr in range(16):
                et_r = etvec[rr]
                r = g * 16 + rr
                acc = jnp.zeros((16,), jnp.float32)
                for k in range(H // 16):
                    t = (psb[r, pl.ds(k * 16, 16)]
                         + pdb[r, pl.ds(k * 16, 16)]
                         + ctv[et_r, pl.ds(k * 16, 16)])
                    t = jnp.maximum(t, 0.0)
                    acc = acc + t * w2s[k]
                lvec = jnp.where(lanes == rr, jnp.sum(acc), lvec)
            v = lvec + b2vec
            outv[pl.ds(j * C + g * 16, 16)] = 1.0 / (1.0 + jnp.exp(-v))
            return _
        lax.fori_loop(0, C // 16, gbody, 0)

    fire(0, ps0, pd0, sem1)

    def pair(p, carry):
        fire(2 * p + 1, ps1, pd1, sem2)
        drain(ps0, pd0, sem1)
        compute(2 * p, ps0, pd0)

        @pl.when(p < nch_c // 2 - 1)
        def _fire_next():
            fire(2 * p + 2, ps0, pd0, sem1)
        drain(ps1, pd1, sem2)
        compute(2 * p + 1, ps1, pd1)
        return carry
    lax.fori_loop(0, nch_c // 2, pair, 0)

    @pl.when(c == 0)
    def _():
        pltpu.sync_copy(outv.at[pl.ds(0, ECH[0] * C)],
                        out_hbm.at[pl.ds(base * C, ECH[0] * C)])

    @pl.when(c == 1)
    def _():
        pltpu.sync_copy(outv.at[pl.ds(0, ECH[1] * C)],
                        out_hbm.at[pl.ds(base * C, ECH[1] * C)])


_edge_sc = pl.kernel(
    _edge_sc_body,
    out_type=[jax.ShapeDtypeStruct((EP,), jnp.float32)],
    mesh=_MESH,
    compiler_params=pltpu.CompilerParams(needs_layout_passes=False),
    scratch_types=[
        pltpu.VMEM((max(ECH), C), jnp.int32),
        pltpu.VMEM((max(ECH), C), jnp.int32),
        pltpu.VMEM((max(ECH), C), jnp.int32),
        pltpu.VMEM((NT, H), jnp.float32),
        pltpu.VMEM((H,), jnp.float32),
        pltpu.VMEM((128,), jnp.float32),
        pltpu.VMEM((C, H), jnp.float32),
        pltpu.VMEM((C, H), jnp.float32),
        pltpu.VMEM((C, H), jnp.float32),
        pltpu.VMEM((C, H), jnp.float32),
        pltpu.VMEM((max(ECH) * C,), jnp.float32),
        pltpu.SemaphoreType.DMA,
        pltpu.SemaphoreType.DMA,
    ],
)


_BR = 1000
_NB = N // _BR


def _tc0_body(x_ref, wt_ref, b_ref, asv_ref, adv_ref, h_ref, asn_ref, adn_ref):
    h = jnp.dot(x_ref[...], wt_ref[...],
                preferred_element_type=jnp.float32) + b_ref[...]
    h_ref[...] = h
    asn_ref[...] = jnp.dot(h, asv_ref[...], preferred_element_type=jnp.float32)
    adn_ref[...] = jnp.dot(h, adv_ref[...], preferred_element_type=jnp.float32)


def _tc0(x, wt, b, asv, adv):
    return pl.pallas_call(
        _tc0_body,
        grid=(_NB,),
        in_specs=[
            pl.BlockSpec((_BR, H), lambda i: (i, 0)),
            pl.BlockSpec((H, H), lambda i: (0, 0)),
            pl.BlockSpec((1, H), lambda i: (0, 0)),
            pl.BlockSpec((H, 1), lambda i: (0, 0)),
            pl.BlockSpec((H, 1), lambda i: (0, 0)),
        ],
        out_specs=[
            pl.BlockSpec((_BR, H), lambda i: (i, 0)),
            pl.BlockSpec((_BR, 1), lambda i: (i, 0)),
            pl.BlockSpec((_BR, 1), lambda i: (i, 0)),
        ],
        out_shape=[
            jax.ShapeDtypeStruct((N, H), jnp.float32),
            jax.ShapeDtypeStruct((N, 1), jnp.float32),
            jax.ShapeDtypeStruct((N, 1), jnp.float32),
        ],
    )(x, wt, b, asv, adv)


def _tclayer_body(a0_ref, a1_ref, s0_ref, s1_ref, wt_ref, b_ref, asv_ref,
                  adv_ref, h_ref, asn_ref, adn_ref):
    inv = 1.0 / (s0_ref[...] + s1_ref[...] + jnp.float32(1e-16))
    x = (a0_ref[...] + a1_ref[...]) * inv
    h = jnp.dot(x, wt_ref[...], preferred_element_type=jnp.float32) + b_ref[...]
    h = jnp.maximum(h, 0.0)
    h_ref[...] = h
    asn_ref[...] = jnp.dot(h, asv_ref[...], preferred_element_type=jnp.float32)
    adn_ref[...] = jnp.dot(h, adv_ref[...], preferred_element_type=jnp.float32)


def _tclayer(a0, a1, s0, s1, wt, b, asv, adv):
    return pl.pallas_call(
        _tclayer_body,
        grid=(_NB,),
        in_specs=[
            pl.BlockSpec((_BR, H), lambda i: (i, 0)),
            pl.BlockSpec((_BR, H), lambda i: (i, 0)),
            pl.BlockSpec((_BR, 1), lambda i: (i, 0)),
            pl.BlockSpec((_BR, 1), lambda i: (i, 0)),
            pl.BlockSpec((H, H), lambda i: (0, 0)),
            pl.BlockSpec((1, H), lambda i: (0, 0)),
            pl.BlockSpec((H, 1), lambda i: (0, 0)),
            pl.BlockSpec((H, 1), lambda i: (0, 0)),
        ],
        out_specs=[
            pl.BlockSpec((_BR, H), lambda i: (i, 0)),
            pl.BlockSpec((_BR, 1), lambda i: (i, 0)),
            pl.BlockSpec((_BR, 1), lambda i: (i, 0)),
        ],
        out_shape=[
            jax.ShapeDtypeStruct((N, H), jnp.float32),
            jax.ShapeDtypeStruct((N, 1), jnp.float32),
            jax.ShapeDtypeStruct((N, 1), jnp.float32),
        ],
    )(a0, a1, s0, s1, wt, b, asv, adv)


def _tcfinal_body(h_ref, w1h_ref, c1_ref, wns2_ref, bns2_ref, wst_ref, wdt_ref,
                  nsc_ref, ps_ref, pd_ref):
    h = h_ref[...]
    nsh = jnp.maximum(
        jnp.dot(h, w1h_ref[...], preferred_element_type=jnp.float32)
        + c1_ref[...], 0.0)
    logit = jnp.dot(nsh, wns2_ref[...],
                    preferred_element_type=jnp.float32) + bns2_ref[...]
    nsc_ref[...] = 1.0 / (1.0 + jnp.exp(-logit))
    ps_ref[...] = jnp.dot(h, wst_ref[...], preferred_element_type=jnp.float32)
    pd_ref[...] = jnp.dot(h, wdt_ref[...], preferred_element_type=jnp.float32)


def _tcfinal(h, w1h, c1, wns2, bns2, wst, wdt):
    return pl.pallas_call(
        _tcfinal_body,
        grid=(_NB,),
        in_specs=[
            pl.BlockSpec((_BR, H), lambda i: (i, 0)),
            pl.BlockSpec((H, H), lambda i: (0, 0)),
            pl.BlockSpec((1, H), lambda i: (0, 0)),
            pl.BlockSpec((H, 1), lambda i: (0, 0)),
            pl.BlockSpec((1, 1), lambda i: (0, 0)),
            pl.BlockSpec((H, H), lambda i: (0, 0)),
            pl.BlockSpec((H, H), lambda i: (0, 0)),
        ],
        out_specs=[
            pl.BlockSpec((_BR, 1), lambda i: (i, 0)),
            pl.BlockSpec((_BR, H), lambda i: (i, 0)),
            pl.BlockSpec((_BR, H), lambda i: (i, 0)),
        ],
        out_shape=[
            jax.ShapeDtypeStruct((N, 1), jnp.float32),
            jax.ShapeDtypeStruct((N, H), jnp.float32),
            jax.ShapeDtypeStruct((N, H), jnp.float32),
        ],
    )(h, w1h, c1, wns2, bns2, wst, wdt)


def _tcprep_body(q_ref, wqt_ref, bq_ref, ed_ref, wst_ref, bs_ref, at_ref,
                 wtes_ref, wqes_ref, bes_ref, w1qt_ref, bns1_ref,
                 qh_ref, te_ref, att_ref, ct_ref, c1_ref):
    qh = jnp.dot(q_ref[...], wqt_ref[...],
                 preferred_element_type=jnp.float32) + bq_ref[...]
    qh_ref[...] = qh
    te = jnp.dot(ed_ref[...], wst_ref[...],
                 preferred_element_type=jnp.float32) + bs_ref[...]
    te_ref[...] = te
    att_ref[...] = jax.lax.dot_general(
        at_ref[...], te, (((1,), (1,)), ((), ())),
        preferred_element_type=jnp.float32)
    ct_ref[...] = (jnp.dot(te, wtes_ref[...], preferred_element_type=jnp.float32)
                   + jnp.dot(qh, wqes_ref[...],
                             preferred_element_type=jnp.float32)
                   + bes_ref[...])
    c1_ref[...] = jnp.dot(qh, w1qt_ref[...],
                          preferred_element_type=jnp.float32) + bns1_ref[...]


def _tcprep(q2, wqt, bq, ed, wst, bs, at, wtes, wqes, bes, w1qt, bns1):
    return pl.pallas_call(
        _tcprep_body,
        out_shape=[
            jax.ShapeDtypeStruct((1, H), jnp.float32),
            jax.ShapeDtypeStruct((NT, 16), jnp.float32),
            jax.ShapeDtypeStruct((L, NT), jnp.float32),
            jax.ShapeDtypeStruct((NT, H), jnp.float32),
            jax.ShapeDtypeStruct((1, H), jnp.float32),
        ],
    )(q2, wqt, bq, ed, wst, bs, at, wtes, wqes, bes, w1qt, bns1)


@jax.jit
def kernel(node_features, edge_index, edge_type, edge_descriptor, query,
           W_node_in, b_node_in, W_query_in, b_query_in, W_schema, b_schema,
           a_src, a_dst, a_type, W_mp, b_mp,
           W_ns1, b_ns1, W_ns2, b_ns2, W_es1, b_es1, W_es2, b_es2):
    src = edge_index[0].astype(jnp.int32)
    dst = edge_index[1].astype(jnp.int32)
    et = edge_type.astype(jnp.int32)
    pad = EP - E
    zpad = jnp.zeros((pad,), jnp.int32)
    src2 = jnp.concatenate([src, zpad]).reshape(EP // C, C)
    et2 = jnp.concatenate([et, zpad]).reshape(EP // C, C)
    dst2s = jnp.concatenate([dst, jnp.full((pad,), N, jnp.int32)]).reshape(
        EP // C, C)
    dst2g = jnp.concatenate([dst, zpad]).reshape(EP // C, C)

    qh, type_emb, att_all, ct, c1 = _tcprep(
        query.reshape(1, H), W_query_in.T, b_query_in.reshape(1, H),
        edge_descriptor, W_schema.T, b_schema.reshape(1, 16),
        a_type, W_es1[:, 2 * H:2 * H + NT].T, W_es1[:, 2 * H + NT:].T,
        b_es1.reshape(1, H), W_ns1[:, H:].T, b_ns1.reshape(1, H))
    att_pad = jnp.pad(att_all, ((0, 0), (0, 128 - NT)))

    h, asn, adn = _tc0(node_features, W_node_in.T, b_node_in.reshape(1, H),
                       a_src[0].reshape(H, 1), a_dst[0].reshape(H, 1))

    npad = jnp.zeros((NP2 - N,), jnp.float32)
    for l in range(L):
        asn_p = jnp.concatenate([asn.reshape(N), npad])
        adn_p = jnp.concatenate([adn.reshape(N), npad])
        s_p, agg_p = _layer_sc(src2, dst2s, et2, h, asn_p, adn_p, att_pad[l])
        nl = min(l + 1, L - 1)
        h, asn, adn = _tclayer(agg_p[0], agg_p[1],
                               s_p[0, :N].reshape(N, 1),
                               s_p[1, :N].reshape(N, 1),
                               W_mp[l].T, b_mp[l].reshape(1, H),
                               a_src[nl].reshape(H, 1), a_dst[nl].reshape(H, 1))

    nscore, ps, pd = _tcfinal(h, W_ns1[:, :H].T, c1, W_ns2.T,
                              b_ns2.reshape(1, 1),
                              W_es1[:, :H].T, W_es1[:, H:2 * H].T)

    b2v = jnp.full((128,), b_es2[0], jnp.float32)
    (escore,) = _edge_sc(src2, dst2g, et2, ps, pd, ct, W_es2[0], b2v)

    return nscore.reshape(N), escore[:E], h, type_emb


# speedup vs baseline: 1.2634x; 1.0383x over previous
import jax
import jax.numpy as jnp
from jax import lax
from jax.experimental import pallas as pl
from jax.experimental.pallas import tpu as pltpu
from jax.experimental.pallas import tpu_sc as plsc

N = 10000
E = 320000
H = 128
NT = 16
L = 3

NC = 2
NS = 16
NW = NC * NS
C = 128
NCH = 80
EPW = NCH * C
EP = NW * EPW
NP2 = 10240
NPAD = N + 16

_MESH = plsc.VectorSubcoreMesh(
    core_axis_name="c", subcore_axis_name="s", num_cores=NC, num_subcores=NS)


GRP = 16
NGRP = NCH // GRP
LCH = (144, 16)
ECH = (96, 64)


def _layer_sc_body(src_hbm, dst_hbm, et_hbm, h_hbm, asn_hbm, adn_hbm, att_hbm,
                   s_out, agg_out,
                   src_g, dst_g, et_g, att_v, asn0, adn0, asn1, adn1,
                   ex0, ex1, row0, row1,
                   zbuf, zvec, shared_s, shared_agg, sem1, sem2):
    c = lax.axis_index("c")
    s = lax.axis_index("s")
    nch_c = jnp.where(c == 0, LCH[0], LCH[1])
    base = jnp.where(c == 0, s * LCH[0], NS * LCH[0] + s * LCH[1])

    for i in range(8):
        for k in range(8):
            zbuf[i, pl.ds(k * 16, 16)] = jnp.zeros((16,), jnp.float32)
    for i in range(40):
        zvec[pl.ds(i * 16, 16)] = jnp.zeros((16,), jnp.float32)

    def zagg(i, _):
        pltpu.sync_copy(zbuf, shared_agg.at[pl.ds(s * 640 + i * 8, 8), :])
        return _
    lax.fori_loop(0, 80, zagg, 0)
    pltpu.sync_copy(zvec, shared_s.at[pl.ds(s * 640, 640)])

    pltpu.sync_copy(att_hbm, att_v)

    plsc.subcore_barrier()

    def fire(jj, row, asn_c, adn_c, sem):
        pltpu.async_copy(h_hbm.at[src_g.at[jj]], row, sem)
        pltpu.async_copy(asn_hbm.at[src_g.at[jj]], asn_c, sem)
        pltpu.async_copy(adn_hbm.at[dst_g.at[jj]], adn_c, sem)

    def drain(row, asn_c, adn_c, sem):
        pltpu.make_async_copy(h_hbm.at[pl.ds(0, C)], row, sem).wait()
        pltpu.make_async_copy(asn_hbm.at[pl.ds(0, C)], asn_c, sem).wait()
        pltpu.make_async_copy(adn_hbm.at[pl.ds(0, C)], adn_c, sem).wait()

    def compute(jj, row, asn_c, adn_c, exv_b):
        for gg in range(C // 16):
            ti = et_g[jj, pl.ds(gg * 16, 16)]
            z = (asn_c[pl.ds(gg * 16, 16)] + adn_c[pl.ds(gg * 16, 16)]
                 + plsc.load_gather(att_v, [ti]))
            e = jnp.where(z >= 0.0, z, z * jnp.float32(0.2))
            exv_b[pl.ds(gg * 16, 16)] = jnp.exp(e)

        def gbody(gg, _):
            exvec = exv_b[pl.ds(gg * 16, 16)]
            for rr in range(16):
                v = jnp.full((16,), exvec[rr], jnp.float32)
                r = gg * 16 + rr
                for k in range(H // 16):
                    row[r, pl.ds(k * 16, 16)] = row[r, pl.ds(k * 16, 16)] * v
            return _
        lax.fori_loop(0, C // 16, gbody, 0)

        idxd = dst_g.at[jj]
        pltpu.sync_copy(row, shared_agg.at[idxd], add=True)
        pltpu.sync_copy(exv_b, shared_s.at[idxd], add=True)

    def group(g, _):
        pltpu.sync_copy(src_hbm.at[pl.ds(base + g * GRP, GRP), :], src_g)
        pltpu.sync_copy(dst_hbm.at[pl.ds(base + g * GRP, GRP), :], dst_g)
        pltpu.sync_copy(et_hbm.at[pl.ds(base + g * GRP, GRP), :], et_g)

        fire(0, row0, asn0, adn0, sem1)

        def pair(p, carry):
            fire(2 * p + 1, row1, asn1, adn1, sem2)
            drain(row0, asn0, adn0, sem1)
            compute(2 * p, row0, asn0, adn0, ex0)

            @pl.when(p < GRP // 2 - 1)
            def _fire_next():
                fire(2 * p + 2, row0, asn0, adn0, sem1)
            drain(row1, asn1, adn1, sem2)
            compute(2 * p + 1, row1, asn1, adn1, ex1)
            return carry
        lax.fori_loop(0, GRP // 2, pair, 0)
        return _
    lax.fori_loop(0, nch_c // GRP, group, 0)

    plsc.subcore_barrier()

    @pl.when(s < 15)
    def _():
        pltpu.sync_copy(shared_agg.at[pl.ds(s * 624, 624), :],
                        agg_out.at[c, pl.ds(s * 624, 624), :])

    @pl.when(s == 15)
    def _():
        pltpu.sync_copy(shared_agg.at[pl.ds(15 * 624, 640), :],
                        agg_out.at[c, pl.ds(15 * 624, 640), :])

    @pl.when(s == 0)
    def _():
        pltpu.sync_copy(shared_s, s_out.at[c])


_layer_sc = pl.kernel(
    _layer_sc_body,
    out_type=[
        jax.ShapeDtypeStruct((NC, NP2), jnp.float32),
        jax.ShapeDtypeStruct((NC, N, H), jnp.float32),
    ],
    mesh=_MESH,
    compiler_params=pltpu.CompilerParams(needs_layout_passes=False),
    scratch_types=[
        pltpu.VMEM((GRP, C), jnp.int32),
        pltpu.VMEM((GRP, C), jnp.int32),
        pltpu.VMEM((GRP, C), jnp.int32),
        pltpu.VMEM((128,), jnp.float32),
        pltpu.VMEM((C,), jnp.float32),
        pltpu.VMEM((C,), jnp.float32),
        pltpu.VMEM((C,), jnp.float32),
        pltpu.VMEM((C,), jnp.float32),
        pltpu.VMEM((C,), jnp.float32),
        pltpu.VMEM((C,), jnp.float32),
        pltpu.VMEM((C, H), jnp.float32),
        pltpu.VMEM((C, H), jnp.float32),
        pltpu.VMEM((8, H), jnp.float32),
        pltpu.VMEM((640,), jnp.float32),
        pltpu.VMEM_SHARED((NP2,), jnp.float32),
        pltpu.VMEM_SHARED((NP2, H), jnp.float32),
        pltpu.SemaphoreType.DMA,
        pltpu.SemaphoreType.DMA,
    ],
)


def _edge_sc_body(src_hbm, dst_hbm, et_hbm, ps_hbm, pd_hbm, ct_hbm, w2_hbm,
                  b2_hbm, out_hbm,
                  srcv, dstv, etv, ctv, w2v, b2v, ps0, pd0, ps1, pd1,
                  outv, sem1, sem2):
    c = lax.axis_index("c")
    s = lax.axis_index("s")
    nch_c = jnp.where(c == 0, ECH[0], ECH[1])
    base = jnp.where(c == 0, s * ECH[0], NS * ECH[0] + s * ECH[1])

    @pl.when(c == 0)
    def _():
        pltpu.sync_copy(src_hbm.at[pl.ds(base, ECH[0]), :],
                        srcv.at[pl.ds(0, ECH[0])])
        pltpu.sync_copy(dst_hbm.at[pl.ds(base, ECH[0]), :],
                        dstv.at[pl.ds(0, ECH[0])])
        pltpu.sync_copy(et_hbm.at[pl.ds(base, ECH[0]), :],
                        etv.at[pl.ds(0, ECH[0])])

    @pl.when(c == 1)
    def _():
        pltpu.sync_copy(src_hbm.at[pl.ds(base, ECH[1]), :],
                        srcv.at[pl.ds(0, ECH[1])])
        pltpu.sync_copy(dst_hbm.at[pl.ds(base, ECH[1]), :],
                        dstv.at[pl.ds(0, ECH[1])])
        pltpu.sync_copy(et_hbm.at[pl.ds(base, ECH[1]), :],
                        etv.at[pl.ds(0, ECH[1])])
    pltpu.sync_copy(ct_hbm, ctv)
    pltpu.sync_copy(w2_hbm, w2v)
    pltpu.sync_copy(b2_hbm, b2v)

    w2s = [w2v[pl.ds(k * 16, 16)] for k in range(H // 16)]
    b2vec = b2v[pl.ds(0, 16)]
    lanes = lax.broadcasted_iota(jnp.int32, (16,), 0)

    def fire(j, psb, pdb, sem):
        pltpu.async_copy(ps_hbm.at[srcv.at[j]], psb, sem)
        pltpu.async_copy(pd_hbm.at[dstv.at[j]], pdb, sem)

    def drain(psb, pdb, sem):
        pltpu.make_async_copy(ps_hbm.at[pl.ds(0, C), :], psb, sem).wait()
        pltpu.make_async_copy(pd_hbm.at[pl.ds(0, C), :], pdb, sem).wait()

    def compute(j, psb, pdb):
        def gbody(g, _):
            etvec = etv[j, pl.ds(g * 16, 16)]
            lvec = jnp.zeros((16,), jnp.float32)
            for rr in range(16):
                et_r = etvec[rr]
                r = g * 16 + rr
                acc = jnp.zeros((16,), jnp.float32)
                for k in range(H // 16):
                    t = (psb[r, pl.ds(k * 16, 16)]
                         + pdb[r, pl.ds(k * 16, 16)]
                         + ctv[et_r, pl.ds(k * 16, 16)])
                    t = jnp.maximum(t, 0.0)
                    acc = acc + t * w2s[k]
                lvec = jnp.where(lanes == rr, jnp.sum(acc), lvec)
            v = lvec + b2vec
            outv[pl.ds(j * C + g * 16, 16)] = 1.0 / (1.0 + jnp.exp(-v))
            return _
        lax.fori_loop(0, C // 16, gbody, 0)

    fire(0, ps0, pd0, sem1)

    def pair(p, carry):
        fire(2 * p + 1, ps1, pd1, sem2)
        drain(ps0, pd0, sem1)
        compute(2 * p, ps0, pd0)

        @pl.when(p < nch_c // 2 - 1)
        def _fire_next():
            fire(2 * p + 2, ps0, pd0, sem1)
        drain(ps1, pd1, sem2)
        compute(2 * p + 1, ps1, pd1)
        return carry
    lax.fori_loop(0, nch_c // 2, pair, 0)

    @pl.when(c == 0)
    def _():
        pltpu.sync_copy(outv.at[pl.ds(0, ECH[0] * C)],
                        out_hbm.at[pl.ds(base * C, ECH[0] * C)])

    @pl.when(c == 1)
    def _():
        pltpu.sync_copy(outv.at[pl.ds(0, ECH[1] * C)],
                        out_hbm.at[pl.ds(base * C, ECH[1] * C)])


_edge_sc = pl.kernel(
    _edge_sc_body,
    out_type=[jax.ShapeDtypeStruct((EP,), jnp.float32)],
    mesh=_MESH,
    compiler_params=pltpu.CompilerParams(needs_layout_passes=False),
    scratch_types=[
        pltpu.VMEM((max(ECH), C), jnp.int32),
        pltpu.VMEM((max(ECH), C), jnp.int32),
        pltpu.VMEM((max(ECH), C), jnp.int32),
        pltpu.VMEM((NT, H), jnp.float32),
        pltpu.VMEM((H,), jnp.float32),
        pltpu.VMEM((128,), jnp.float32),
        pltpu.VMEM((C, H), jnp.float32),
        pltpu.VMEM((C, H), jnp.float32),
        pltpu.VMEM((C, H), jnp.float32),
        pltpu.VMEM((C, H), jnp.float32),
        pltpu.VMEM((max(ECH) * C,), jnp.float32),
        pltpu.SemaphoreType.DMA,
        pltpu.SemaphoreType.DMA,
    ],
)


_BR = 1000
_NB = N // _BR


def _tc0_body(x_ref, wt_ref, b_ref, asv_ref, adv_ref, h_ref, asn_ref, adn_ref):
    h = jnp.dot(x_ref[...], wt_ref[...],
                preferred_element_type=jnp.float32) + b_ref[...]
    h_ref[...] = h
    asn_ref[...] = jnp.dot(h, asv_ref[...], preferred_element_type=jnp.float32)
    adn_ref[...] = jnp.dot(h, adv_ref[...], preferred_element_type=jnp.float32)


def _tc0(x, wt, b, asv, adv):
    return pl.pallas_call(
        _tc0_body,
        grid=(_NB,),
        in_specs=[
            pl.BlockSpec((_BR, H), lambda i: (i, 0)),
            pl.BlockSpec((H, H), lambda i: (0, 0)),
            pl.BlockSpec((1, H), lambda i: (0, 0)),
            pl.BlockSpec((H, 1), lambda i: (0, 0)),
            pl.BlockSpec((H, 1), lambda i: (0, 0)),
        ],
        out_specs=[
            pl.BlockSpec((_BR, H), lambda i: (i, 0)),
            pl.BlockSpec((_BR, 1), lambda i: (i, 0)),
            pl.BlockSpec((_BR, 1), lambda i: (i, 0)),
        ],
        out_shape=[
            jax.ShapeDtypeStruct((N, H), jnp.float32),
            jax.ShapeDtypeStruct((N, 1), jnp.float32),
            jax.ShapeDtypeStruct((N, 1), jnp.float32),
        ],
    )(x, wt, b, asv, adv)


def _tclayer_body(a0_ref, a1_ref, s0_ref, s1_ref, wt_ref, b_ref, asv_ref,
                  adv_ref, h_ref, asn_ref, adn_ref):
    inv = 1.0 / (s0_ref[...] + s1_ref[...] + jnp.float32(1e-16))
    x = (a0_ref[...] + a1_ref[...]) * inv
    h = jnp.dot(x, wt_ref[...], preferred_element_type=jnp.float32) + b_ref[...]
    h = jnp.maximum(h, 0.0)
    h_ref[...] = h
    asn_ref[...] = jnp.dot(h, asv_ref[...], preferred_element_type=jnp.float32)
    adn_ref[...] = jnp.dot(h, adv_ref[...], preferred_element_type=jnp.float32)


def _tclayer(a0, a1, s0, s1, wt, b, asv, adv):
    return pl.pallas_call(
        _tclayer_body,
        grid=(_NB,),
        in_specs=[
            pl.BlockSpec((_BR, H), lambda i: (i, 0)),
            pl.BlockSpec((_BR, H), lambda i: (i, 0)),
            pl.BlockSpec((_BR, 1), lambda i: (i, 0)),
            pl.BlockSpec((_BR, 1), lambda i: (i, 0)),
            pl.BlockSpec((H, H), lambda i: (0, 0)),
            pl.BlockSpec((1, H), lambda i: (0, 0)),
            pl.BlockSpec((H, 1), lambda i: (0, 0)),
            pl.BlockSpec((H, 1), lambda i: (0, 0)),
        ],
        out_specs=[
            pl.BlockSpec((_BR, H), lambda i: (i, 0)),
            pl.BlockSpec((_BR, 1), lambda i: (i, 0)),
            pl.BlockSpec((_BR, 1), lambda i: (i, 0)),
        ],
        out_shape=[
            jax.ShapeDtypeStruct((N, H), jnp.float32),
            jax.ShapeDtypeStruct((N, 1), jnp.float32),
            jax.ShapeDtypeStruct((N, 1), jnp.float32),
        ],
    )(a0, a1, s0, s1, wt, b, asv, adv)


def _tcfinal_body(h_ref, w1h_ref, c1_ref, wns2_ref, bns2_ref, wst_ref, wdt_ref,
                  nsc_ref, ps_ref, pd_ref):
    h = h_ref[...]
    nsh = jnp.maximum(
        jnp.dot(h, w1h_ref[...], preferred_element_type=jnp.float32)
        + c1_ref[...], 0.0)
    logit = jnp.dot(nsh, wns2_ref[...],
                    preferred_element_type=jnp.float32) + bns2_ref[...]
    nsc_ref[...] = 1.0 / (1.0 + jnp.exp(-logit))
    ps_ref[...] = jnp.dot(h, wst_ref[...], preferred_element_type=jnp.float32)
    pd_ref[...] = jnp.dot(h, wdt_ref[...], preferred_element_type=jnp.float32)


def _tcfinal(h, w1h, c1, wns2, bns2, wst, wdt):
    return pl.pallas_call(
        _tcfinal_body,
        grid=(_NB,),
        in_specs=[
            pl.BlockSpec((_BR, H), lambda i: (i, 0)),
            pl.BlockSpec((H, H), lambda i: (0, 0)),
            pl.BlockSpec((1, H), lambda i: (0, 0)),
            pl.BlockSpec((H, 1), lambda i: (0, 0)),
            pl.BlockSpec((1, 1), lambda i: (0, 0)),
            pl.BlockSpec((H, H), lambda i: (0, 0)),
            pl.BlockSpec((H, H), lambda i: (0, 0)),
        ],
        out_specs=[
            pl.BlockSpec((_BR, 1), lambda i: (i, 0)),
            pl.BlockSpec((_BR, H), lambda i: (i, 0)),
            pl.BlockSpec((_BR, H), lambda i: (i, 0)),
        ],
        out_shape=[
            jax.ShapeDtypeStruct((N, 1), jnp.float32),
            jax.ShapeDtypeStruct((N, H), jnp.float32),
            jax.ShapeDtypeStruct((N, H), jnp.float32),
        ],
    )(h, w1h, c1, wns2, bns2, wst, wdt)


def _tcprep_body(q_ref, wqt_ref, bq_ref, ed_ref, wst_ref, bs_ref, at_ref,
                 wtes_ref, wqes_ref, bes_ref, w1qt_ref, bns1_ref,
                 qh_ref, te_ref, att_ref, ct_ref, c1_ref):
    qh = jnp.dot(q_ref[...], wqt_ref[...],
                 preferred_element_type=jnp.float32) + bq_ref[...]
    qh_ref[...] = qh
    te = jnp.dot(ed_ref[...], wst_ref[...],
                 preferred_element_type=jnp.float32) + bs_ref[...]
    te_ref[...] = te
    att_ref[...] = jax.lax.dot_general(
        at_ref[...], te, (((1,), (1,)), ((), ())),
        preferred_element_type=jnp.float32)
    ct_ref[...] = (jnp.dot(te, wtes_ref[...], preferred_element_type=jnp.float32)
                   + jnp.dot(qh, wqes_ref[...],
                             preferred_element_type=jnp.float32)
                   + bes_ref[...])
    c1_ref[...] = jnp.dot(qh, w1qt_ref[...],
                          preferred_element_type=jnp.float32) + bns1_ref[...]


def _tcprep(q2, wqt, bq, ed, wst, bs, at, wtes, wqes, bes, w1qt, bns1):
    return pl.pallas_call(
        _tcprep_body,
        out_shape=[
            jax.ShapeDtypeStruct((1, H), jnp.float32),
            jax.ShapeDtypeStruct((NT, 16), jnp.float32),
            jax.ShapeDtypeStruct((L, NT), jnp.float32),
            jax.ShapeDtypeStruct((NT, H), jnp.float32),
            jax.ShapeDtypeStruct((1, H), jnp.float32),
        ],
    )(q2, wqt, bq, ed, wst, bs, at, wtes, wqes, bes, w1qt, bns1)


@jax.jit
def kernel(node_features, edge_index, edge_type, edge_descriptor, query,
           W_node_in, b_node_in, W_query_in, b_query_in, W_schema, b_schema,
           a_src, a_dst, a_type, W_mp, b_mp,
           W_ns1, b_ns1, W_ns2, b_ns2, W_es1, b_es1, W_es2, b_es2):
    src = edge_index[0].astype(jnp.int32)
    dst = edge_index[1].astype(jnp.int32)
    et = edge_type.astype(jnp.int32)
    pad = EP - E
    zpad = jnp.zeros((pad,), jnp.int32)
    src2 = jnp.concatenate([src, zpad]).reshape(EP // C, C)
    et2 = jnp.concatenate([et, zpad]).reshape(EP // C, C)
    dst2s = jnp.concatenate([dst, jnp.full((pad,), N, jnp.int32)]).reshape(
        EP // C, C)
    dst2g = jnp.concatenate([dst, zpad]).reshape(EP // C, C)

    qh, type_emb, att_all, ct, c1 = _tcprep(
        query.reshape(1, H), W_query_in.T, b_query_in.reshape(1, H),
        edge_descriptor, W_schema.T, b_schema.reshape(1, 16),
        a_type, W_es1[:, 2 * H:2 * H + NT].T, W_es1[:, 2 * H + NT:].T,
        b_es1.reshape(1, H), W_ns1[:, H:].T, b_ns1.reshape(1, H))
    att_pad = jnp.pad(att_all, ((0, 0), (0, 128 - NT)))

    h, asn, adn = _tc0(node_features, W_node_in.T, b_node_in.reshape(1, H),
                       a_src[0].reshape(H, 1), a_dst[0].reshape(H, 1))

    npad = jnp.zeros((NP2 - N,), jnp.float32)
    for l in range(L):
        asn_p = jnp.concatenate([asn.reshape(N), npad])
        adn_p = jnp.concatenate([adn.reshape(N), npad])
        s_p, agg_p = _layer_sc(src2, dst2s, et2, h, asn_p, adn_p, att_pad[l])
        nl = min(l + 1, L - 1)
        h, asn, adn = _tclayer(agg_p[0], agg_p[1],
                               s_p[0, :N].reshape(N, 1),
                               s_p[1, :N].reshape(N, 1),
                               W_mp[l].T, b_mp[l].reshape(1, H),
                               a_src[nl].reshape(H, 1), a_dst[nl].reshape(H, 1))

    nscore, ps, pd = _tcfinal(h, W_ns1[:, :H].T, c1, W_ns2.T,
                              b_ns2.reshape(1, 1),
                              W_es1[:, :H].T, W_es1[:, H:2 * H].T)

    b2v = jnp.full((128,), b_es2[0], jnp.float32)
    (escore,) = _edge_sc(src2, dst2g, et2, ps, pd, ct, W_es2[0], b2v)

    return nscore.reshape(N), escore[:E], h, type_emb
